# Initial kernel scaffold; baseline (speedup 1.0000x reference)
import jax
import jax.numpy as jnp
from jax.experimental import pallas as pl


def kernel(x, branch1x1_w, branch1x1_b, branch3x3_1_w, branch3x3_1_b, branch3x3_2a_w, branch3x3_2a_b, branch3x3_2b_w, branch3x3_2b_b, branch3x3dbl_1_w, branch3x3dbl_1_b, branch3x3dbl_2_w, branch3x3dbl_2_b, branch3x3dbl_3a_w, branch3x3dbl_3a_b, branch3x3dbl_3b_w, branch3x3dbl_3b_b, branch_pool_w, branch_pool_b):
    raise NotImplementedError("write your pallas kernel here")


# speedup vs baseline: 1.9421x; 1.9421x over previous
import jax
import jax.numpy as jnp
from jax.experimental import pallas as pl
from jax.experimental.pallas import tpu as pltpu

_VMEM_LIMIT = 48 * 1024 * 1024
_BLOCK_N = 8


def _pad_hw1(y):
    B, H, W, C = y.shape
    zw = jnp.zeros((B, H, 1, C), y.dtype)
    t = jnp.concatenate([zw, y, zw], axis=2)
    zh = jnp.zeros((B, 1, W + 2, C), y.dtype)
    return jnp.concatenate([zh, t, zh], axis=1)


def _pair(y4, wa_ref, ba_ref, wb_ref, bb_ref):
    B, H, W, C = y4.shape
    M = B * H * W
    yp_ = _pad_hw1(y4)
    la = jnp.concatenate(
        [yp_[:, 1:1 + H, k:k + W, :].reshape(M, C) for k in range(3)], axis=1)
    a = jnp.maximum(
        jnp.dot(la, wa_ref[...], preferred_element_type=jnp.float32)
        + ba_ref[...], 0.0)
    lb = jnp.concatenate(
        [yp_[:, k:k + H, 1:1 + W, :].reshape(M, C) for k in range(3)], axis=1)
    b = jnp.maximum(
        jnp.dot(lb, wb_ref[...], preferred_element_type=jnp.float32)
        + bb_ref[...], 0.0)
    return a, b


def _fused_kernel(x_ref, ws_ref, b3_ref, b1_ref, bd_ref, bp_ref,
                  w2_ref, b2_ref,
                  wa1_ref, ba1_ref, wb1_ref, bb1_ref,
                  wa2_ref, ba2_ref, wb2_ref, bb2_ref,
                  o_ref):
    B, H, W, Cin = x_ref.shape
    M = B * H * W
    c3 = b3_ref.shape[1]
    c1 = b1_ref.shape[1]
    cd = bd_ref.shape[1]
    cp = bp_ref.shape[1]
    c2 = b2_ref.shape[1]

    x = x_ref[...].reshape(M, Cin)

    z = jnp.dot(x, ws_ref[...], preferred_element_type=jnp.float32)
    y3 = jnp.maximum(z[:, 0:c3] + b3_ref[...], 0.0)
    y1 = jnp.maximum(z[:, c3:c3 + c1] + b1_ref[...], 0.0)
    yd = jnp.maximum(z[:, c3 + c1:c3 + c1 + cd] + bd_ref[...], 0.0)
    zp = z[:, c3 + c1 + cd:c3 + c1 + cd + cp]

    zpp = _pad_hw1(zp.reshape(B, H, W, cp))
    hs = zpp[:, :, 0:W, :] + zpp[:, :, 1:W + 1, :] + zpp[:, :, 2:W + 2, :]
    vs = hs[:, 0:H] + hs[:, 1:H + 1] + hs[:, 2:H + 2]
    yp = jnp.maximum(vs.reshape(M, cp) * (1.0 / 9.0) + bp_ref[...], 0.0)

    a1, b1_ = _pair(y3.astype(jnp.bfloat16).reshape(B, H, W, c3),
                    wa1_ref, ba1_ref, wb1_ref, bb1_ref)

    ydp = _pad_hw1(yd.astype(jnp.bfloat16).reshape(B, H, W, cd))
    l2 = jnp.concatenate(
        [ydp[:, kh:kh + H, kw:kw + W, :].reshape(M, cd)
         for kh in range(3) for kw in range(3)], axis=1)
    t = jnp.maximum(
        jnp.dot(l2, w2_ref[...], preferred_element_type=jnp.float32)
        + b2_ref[...], 0.0)
    a2, b2_ = _pair(t.astype(jnp.bfloat16).reshape(B, H, W, c2),
                    wa2_ref, ba2_ref, wb2_ref, bb2_ref)

    out = jnp.concatenate([y1, a1, b1_, a2, b2_, yp], axis=1)
    o_ref[...] = out.reshape(B, H, W, out.shape[1]).astype(o_ref.dtype)


def kernel(x, branch1x1_w, branch1x1_b, branch3x3_1_w, branch3x3_1_b,
           branch3x3_2a_w, branch3x3_2a_b, branch3x3_2b_w, branch3x3_2b_b,
           branch3x3dbl_1_w, branch3x3dbl_1_b, branch3x3dbl_2_w,
           branch3x3dbl_2_b, branch3x3dbl_3a_w, branch3x3dbl_3a_b,
           branch3x3dbl_3b_w, branch3x3dbl_3b_b, branch_pool_w,
           branch_pool_b):
    N, Cin, H, W = x.shape
    bf16 = jnp.bfloat16
    xh = jnp.transpose(x, (0, 2, 3, 1)).astype(bf16)

    c3 = branch3x3_1_w.shape[1]
    c1 = branch1x1_w.shape[1]
    cd = branch3x3dbl_1_w.shape[1]
    cp = branch_pool_w.shape[1]
    c2 = branch3x3dbl_2_w.shape[2]
    ca = branch3x3_2a_w.shape[2]

    ws = jnp.concatenate(
        [branch3x3_1_w, branch1x1_w, branch3x3dbl_1_w, branch_pool_w],
        axis=1).astype(bf16)
    w2 = branch3x3dbl_2_w.reshape(-1, c2).astype(bf16)
    wa1 = branch3x3_2a_w.reshape(-1, ca).astype(bf16)
    wb1 = branch3x3_2b_w.reshape(-1, ca).astype(bf16)
    wa2 = branch3x3dbl_3a_w.reshape(-1, ca).astype(bf16)
    wb2 = branch3x3dbl_3b_w.reshape(-1, ca).astype(bf16)

    def b2d(b):
        return b.reshape(1, b.shape[-1])

    cout = c1 + 4 * ca + cp
    B = _BLOCK_N

    def wspec(w):
        return pl.BlockSpec(w.shape, lambda n: (0,) * w.ndim)

    args = [ws, b2d(branch3x3_1_b), b2d(branch1x1_b), b2d(branch3x3dbl_1_b),
            b2d(branch_pool_b), w2, b2d(branch3x3dbl_2_b),
            wa1, b2d(branch3x3_2a_b), wb1, b2d(branch3x3_2b_b),
            wa2, b2d(branch3x3dbl_3a_b), wb2, b2d(branch3x3dbl_3b_b)]

    out = pl.pallas_call(
        _fused_kernel,
        out_shape=jax.ShapeDtypeStruct((N, H, W, cout), x.dtype),
        grid=(N // B,),
        in_specs=[pl.BlockSpec((B, H, W, Cin), lambda n: (n, 0, 0, 0))]
        + [wspec(a) for a in args],
        out_specs=pl.BlockSpec((B, H, W, cout), lambda n: (n, 0, 0, 0)),
        compiler_params=pltpu.CompilerParams(
            dimension_semantics=("parallel",),
            vmem_limit_bytes=_VMEM_LIMIT),
    )(xh, *args)

    return jnp.transpose(out, (0, 3, 1, 2))


# speedup vs baseline: 2.5387x; 1.3071x over previous
import jax
import jax.numpy as jnp
from jax.experimental import pallas as pl
from jax.experimental.pallas import tpu as pltpu

_VMEM_LIMIT = 56 * 1024 * 1024
_BLOCK_N = 8


def _pad_hw1(y):
    B, H, W, C = y.shape
    zw = jnp.zeros((B, H, 1, C), y.dtype)
    t = jnp.concatenate([zw, y, zw], axis=2)
    zh = jnp.zeros((B, 1, W + 2, C), y.dtype)
    return jnp.concatenate([zh, t, zh], axis=1)


def _wcast(w_ref):
    w = w_ref[...].astype(jnp.bfloat16)
    if w.ndim == 3:
        w = w.reshape(w.shape[0] * w.shape[1], w.shape[2])
    return w


def _pair(y4, wa_ref, ba_ref, wb_ref, bb_ref):
    B, H, W, C = y4.shape
    M = B * H * W
    yp_ = _pad_hw1(y4)
    la = jnp.concatenate(
        [yp_[:, 1:1 + H, k:k + W, :].reshape(M, C) for k in range(3)], axis=1)
    a = jnp.maximum(
        jnp.dot(la, _wcast(wa_ref), preferred_element_type=jnp.float32)
        + ba_ref[0], 0.0)
    lb = jnp.concatenate(
        [yp_[:, k:k + H, 1:1 + W, :].reshape(M, C) for k in range(3)], axis=1)
    b = jnp.maximum(
        jnp.dot(lb, _wcast(wb_ref), preferred_element_type=jnp.float32)
        + bb_ref[0], 0.0)
    return a, b


def _fused_kernel(x_ref, w1_ref, b1_ref, w3_ref, b3_ref,
                  wa1_ref, ba1_ref, wb1_ref, bb1_ref,
                  wd_ref, bd_ref, w2_ref, b2_ref,
                  wa2_ref, ba2_ref, wb2_ref, bb2_ref,
                  wp_ref, bp_ref, o_ref):
    B, H, W, Cin = x_ref.shape
    M = B * H * W
    cp = wp_ref.shape[1]
    cd = wd_ref.shape[1]
    c2 = w2_ref.shape[2]

    x = x_ref[...].reshape(M, Cin)

    y3 = jnp.maximum(
        jnp.dot(x, _wcast(w3_ref), preferred_element_type=jnp.float32)
        + b3_ref[0], 0.0)
    y1 = jnp.maximum(
        jnp.dot(x, _wcast(w1_ref), preferred_element_type=jnp.float32)
        + b1_ref[0], 0.0)
    yd = jnp.maximum(
        jnp.dot(x, _wcast(wd_ref), preferred_element_type=jnp.float32)
        + bd_ref[0], 0.0)
    zp = jnp.dot(x, _wcast(wp_ref), preferred_element_type=jnp.float32)

    zpp = _pad_hw1(zp.reshape(B, H, W, cp))
    hs = zpp[:, :, 0:W, :] + zpp[:, :, 1:W + 1, :] + zpp[:, :, 2:W + 2, :]
    vs = hs[:, 0:H] + hs[:, 1:H + 1] + hs[:, 2:H + 2]
    yp = jnp.maximum(vs.reshape(M, cp) * (1.0 / 9.0) + bp_ref[0], 0.0)

    c3 = y3.shape[1]
    a1, b1_ = _pair(y3.astype(jnp.bfloat16).reshape(B, H, W, c3),
                    wa1_ref, ba1_ref, wb1_ref, bb1_ref)

    ydp = _pad_hw1(yd.astype(jnp.bfloat16).reshape(B, H, W, cd))
    l2 = jnp.concatenate(
        [ydp[:, kh:kh + H, kw:kw + W, :].reshape(M, cd)
         for kh in range(3) for kw in range(3)], axis=1)
    t = jnp.maximum(
        jnp.dot(l2, _wcast(w2_ref), preferred_element_type=jnp.float32)
        + b2_ref[0], 0.0)
    a2, b2_ = _pair(t.astype(jnp.bfloat16).reshape(B, H, W, c2),
                    wa2_ref, ba2_ref, wb2_ref, bb2_ref)

    out = jnp.concatenate([y1, a1, b1_, a2, b2_, yp], axis=1)
    o_ref[...] = out.reshape(B, H, W, out.shape[1]).astype(o_ref.dtype)


def kernel(x, branch1x1_w, branch1x1_b, branch3x3_1_w, branch3x3_1_b,
           branch3x3_2a_w, branch3x3_2a_b, branch3x3_2b_w, branch3x3_2b_b,
           branch3x3dbl_1_w, branch3x3dbl_1_b, branch3x3dbl_2_w,
           branch3x3dbl_2_b, branch3x3dbl_3a_w, branch3x3dbl_3a_b,
           branch3x3dbl_3b_w, branch3x3dbl_3b_b, branch_pool_w,
           branch_pool_b):
    N, Cin, H, W = x.shape
    xh = jnp.transpose(x, (0, 2, 3, 1)).astype(jnp.bfloat16)

    cout = (branch1x1_w.shape[1] + 4 * branch3x3_2a_w.shape[2]
            + branch_pool_w.shape[1])
    B = _BLOCK_N

    def wspec(w):
        return pl.BlockSpec(w.shape, lambda n: (0,) * w.ndim)

    args = [branch1x1_w, branch1x1_b, branch3x3_1_w, branch3x3_1_b,
            branch3x3_2a_w, branch3x3_2a_b, branch3x3_2b_w, branch3x3_2b_b,
            branch3x3dbl_1_w, branch3x3dbl_1_b, branch3x3dbl_2_w,
            branch3x3dbl_2_b, branch3x3dbl_3a_w, branch3x3dbl_3a_b,
            branch3x3dbl_3b_w, branch3x3dbl_3b_b, branch_pool_w,
            branch_pool_b]

    out = pl.pallas_call(
        _fused_kernel,
        out_shape=jax.ShapeDtypeStruct((N, H, W, cout), x.dtype),
        grid=(N // B,),
        in_specs=[pl.BlockSpec((B, H, W, Cin), lambda n: (n, 0, 0, 0))]
        + [wspec(a) for a in args],
        out_specs=pl.BlockSpec((B, H, W, cout), lambda n: (n, 0, 0, 0)),
        compiler_params=pltpu.CompilerParams(
            dimension_semantics=("parallel",),
            vmem_limit_bytes=_VMEM_LIMIT),
    )(xh, *args)

    return jnp.transpose(out, (0, 3, 1, 2))


# speedup vs baseline: 3.3706x; 1.3277x over previous
import jax
import jax.numpy as jnp
from jax.experimental import pallas as pl
from jax.experimental.pallas import tpu as pltpu

_VMEM_LIMIT = 56 * 1024 * 1024
_BLOCK_N = 8


def _pad_hw1(y):
    B, H, W, C = y.shape
    zw = jnp.zeros((B, H, 1, C), y.dtype)
    t = jnp.concatenate([zw, y, zw], axis=2)
    zh = jnp.zeros((B, 1, W + 2, C), y.dtype)
    return jnp.concatenate([zh, t, zh], axis=1)


def _wcast(w_ref):
    w = w_ref[...].astype(jnp.bfloat16)
    if w.ndim == 3:
        w = w.reshape(w.shape[0] * w.shape[1], w.shape[2])
    return w


def _pair(y4, wa_ref, ba_ref, wb_ref, bb_ref):
    B, H, W, C = y4.shape
    M = B * H * W
    yp_ = _pad_hw1(y4)
    la = jnp.concatenate(
        [yp_[:, 1:1 + H, k:k + W, :].reshape(M, C) for k in range(3)], axis=1)
    a = jnp.maximum(
        jnp.dot(la, _wcast(wa_ref), preferred_element_type=jnp.float32)
        + ba_ref[0], 0.0)
    lb = jnp.concatenate(
        [yp_[:, k:k + H, 1:1 + W, :].reshape(M, C) for k in range(3)], axis=1)
    b = jnp.maximum(
        jnp.dot(lb, _wcast(wb_ref), preferred_element_type=jnp.float32)
        + bb_ref[0], 0.0)
    return a, b


def _fused_kernel(x_ref, w1_ref, b1_ref, w3_ref, b3_ref,
                  wa1_ref, ba1_ref, wb1_ref, bb1_ref,
                  wd_ref, bd_ref, w2_ref, b2_ref,
                  wa2_ref, ba2_ref, wb2_ref, bb2_ref,
                  wp_ref, bp_ref, o_ref):
    B, H, W, Cin = x_ref.shape
    M = B * H * W
    cp = wp_ref.shape[0]
    cd = wd_ref.shape[0]
    c2 = w2_ref.shape[2]

    x = x_ref[...].reshape(M, Cin).astype(jnp.bfloat16)

    def _stem(wt_ref):
        return jax.lax.dot_general(
            x, wt_ref[...].astype(jnp.bfloat16),
            dimension_numbers=(((1,), (1,)), ((), ())),
            preferred_element_type=jnp.float32)

    y3 = jnp.maximum(_stem(w3_ref) + b3_ref[0], 0.0)
    y1 = jnp.maximum(_stem(w1_ref) + b1_ref[0], 0.0)
    yd = jnp.maximum(_stem(wd_ref) + bd_ref[0], 0.0)
    zp = _stem(wp_ref)

    zpp = _pad_hw1(zp.reshape(B, H, W, cp))
    hs = zpp[:, :, 0:W, :] + zpp[:, :, 1:W + 1, :] + zpp[:, :, 2:W + 2, :]
    vs = hs[:, 0:H] + hs[:, 1:H + 1] + hs[:, 2:H + 2]
    yp = jnp.maximum(vs.reshape(M, cp) * (1.0 / 9.0) + bp_ref[0], 0.0)

    c3 = y3.shape[1]
    a1, b1_ = _pair(y3.astype(jnp.bfloat16).reshape(B, H, W, c3),
                    wa1_ref, ba1_ref, wb1_ref, bb1_ref)

    ydp = _pad_hw1(yd.astype(jnp.bfloat16).reshape(B, H, W, cd))
    l2 = jnp.concatenate(
        [ydp[:, kh:kh + H, kw:kw + W, :].reshape(M, cd)
         for kh in range(3) for kw in range(3)], axis=1)
    t = jnp.maximum(
        jnp.dot(l2, _wcast(w2_ref), preferred_element_type=jnp.float32)
        + b2_ref[0], 0.0)
    a2, b2_ = _pair(t.astype(jnp.bfloat16).reshape(B, H, W, c2),
                    wa2_ref, ba2_ref, wb2_ref, bb2_ref)

    out = jnp.concatenate([y1, a1, b1_, a2, b2_, yp], axis=1)
    o_ref[...] = out.reshape(B, H, W, out.shape[1]).astype(o_ref.dtype)


def kernel(x, branch1x1_w, branch1x1_b, branch3x3_1_w, branch3x3_1_b,
           branch3x3_2a_w, branch3x3_2a_b, branch3x3_2b_w, branch3x3_2b_b,
           branch3x3dbl_1_w, branch3x3dbl_1_b, branch3x3dbl_2_w,
           branch3x3dbl_2_b, branch3x3dbl_3a_w, branch3x3dbl_3a_b,
           branch3x3dbl_3b_w, branch3x3dbl_3b_b, branch_pool_w,
           branch_pool_b):
    N, Cin, H, W = x.shape
    xh = jnp.transpose(x, (0, 2, 3, 1))

    cout = (branch1x1_w.shape[1] + 4 * branch3x3_2a_w.shape[2]
            + branch_pool_w.shape[1])
    B = _BLOCK_N

    branch1x1_w = branch1x1_w.T
    branch3x3_1_w = branch3x3_1_w.T
    branch3x3dbl_1_w = branch3x3dbl_1_w.T
    branch_pool_w = branch_pool_w.T

    def wspec(w):
        return pl.BlockSpec(w.shape, lambda n: (0,) * w.ndim)

    args = [branch1x1_w, branch1x1_b, branch3x3_1_w, branch3x3_1_b,
            branch3x3_2a_w, branch3x3_2a_b, branch3x3_2b_w, branch3x3_2b_b,
            branch3x3dbl_1_w, branch3x3dbl_1_b, branch3x3dbl_2_w,
            branch3x3dbl_2_b, branch3x3dbl_3a_w, branch3x3dbl_3a_b,
            branch3x3dbl_3b_w, branch3x3dbl_3b_b, branch_pool_w,
            branch_pool_b]

    out = pl.pallas_call(
        _fused_kernel,
        out_shape=jax.ShapeDtypeStruct((N, H, W, cout), x.dtype),
        grid=(N // B,),
        in_specs=[pl.BlockSpec((B, H, W, Cin), lambda n: (n, 0, 0, 0))]
        + [wspec(a) for a in args],
        out_specs=pl.BlockSpec((B, H, W, cout), lambda n: (n, 0, 0, 0)),
        compiler_params=pltpu.CompilerParams(
            dimension_semantics=("parallel",),
            vmem_limit_bytes=_VMEM_LIMIT),
    )(xh, *args)

    return jnp.transpose(out, (0, 3, 1, 2))
